# Initial kernel scaffold; baseline (speedup 1.0000x reference)
import jax
import jax.numpy as jnp
from jax.experimental import pallas as pl


def kernel(x, edge_index, W_in, b_in, W_msg, b_msg):
    raise NotImplementedError("write your pallas kernel here")


# speedup vs baseline: 4.7167x; 4.7167x over previous
import functools

import jax
import jax.numpy as jnp
from jax import lax
from jax.experimental import pallas as pl
from jax.experimental.pallas import tpu as pltpu
from jax.experimental.pallas import tpu_sc as plsc

NC = 2
NS = 16
NW = NC * NS
CH = 128
LANES = 16


def _proj_body(x_ref, w_in_ref, b_in_ref, w1_ref, w2_ref, bm_ref, a_ref, b_ref):
    h = jnp.dot(x_ref[...], w_in_ref[...], preferred_element_type=jnp.float32)
    h = jnp.maximum(h + b_in_ref[...], 0.0)
    a_ref[...] = jnp.dot(h, w1_ref[...], preferred_element_type=jnp.float32)
    b_ref[...] = (
        jnp.dot(h, w2_ref[...], preferred_element_type=jnp.float32) + bm_ref[...]
    )


@functools.partial(jax.jit, static_argnames=("blk",))
def _proj(x, w_in, b_in, w1, w2, bm, blk=1000):
    n, d = x.shape
    hid = w_in.shape[1]
    grid = n // blk
    return pl.pallas_call(
        _proj_body,
        grid=(grid,),
        in_specs=[
            pl.BlockSpec((blk, d), lambda i: (i, 0)),
            pl.BlockSpec((d, hid), lambda i: (0, 0)),
            pl.BlockSpec((1, hid), lambda i: (0, 0)),
            pl.BlockSpec((hid, hid), lambda i: (0, 0)),
            pl.BlockSpec((hid, hid), lambda i: (0, 0)),
            pl.BlockSpec((1, hid), lambda i: (0, 0)),
        ],
        out_specs=[
            pl.BlockSpec((blk, hid), lambda i: (i, 0)),
            pl.BlockSpec((blk, hid), lambda i: (i, 0)),
        ],
        out_shape=[
            jax.ShapeDtypeStruct((n, hid), jnp.float32),
            jax.ShapeDtypeStruct((n, hid), jnp.float32),
        ],
    )(x, w_in, b_in.reshape(1, hid), w1, w2, bm.reshape(1, hid))


def _make_sc(epw, nchunks, npad, hid):
    rows_per_tile = npad // NS
    vpr = hid // LANES

    mesh = plsc.VectorSubcoreMesh(core_axis_name="c", subcore_axis_name="s")

    @functools.partial(
        pl.kernel,
        out_type=jax.ShapeDtypeStruct((NC, npad, hid), jnp.float32),
        mesh=mesh,
        scratch_types=[
            pltpu.VMEM((CH,), jnp.int32),
            pltpu.VMEM((CH,), jnp.int32),
            pltpu.VMEM((CH, hid), jnp.float32),
            pltpu.VMEM((CH, hid), jnp.float32),
            pltpu.VMEM_SHARED((npad, hid), jnp.float32),
            pltpu.SemaphoreType.DMA,
            pltpu.SemaphoreType.DMA,
        ],
    )
    def sc_edges(a_hbm, b_hbm, src_hbm, dst_hbm, out_hbm,
                 sidx, didx, a_buf, b_buf, acc, sem_a, sem_b):
        cid = lax.axis_index("c")
        sid = lax.axis_index("s")
        wid = sid * NC + cid

        zero = jnp.zeros((LANES,), jnp.float32)

        def _zero_row(r, _):
            for c in range(vpr):
                a_buf[r, pl.ds(c * LANES, LANES)] = zero
            return 0

        lax.fori_loop(0, CH, _zero_row, 0)

        def _zero_acc(i, _):
            pltpu.sync_copy(a_buf, acc.at[pl.ds(sid * rows_per_tile + i * CH, CH)])
            return 0

        lax.fori_loop(0, rows_per_tile // CH, _zero_acc, 0)
        plsc.subcore_barrier()

        def _chunk(i, _):
            base = wid * epw + i * CH
            pltpu.sync_copy(src_hbm.at[pl.ds(base, CH)], sidx)
            pltpu.sync_copy(dst_hbm.at[pl.ds(base, CH)], didx)
            ca = pltpu.async_copy(a_hbm.at[sidx], a_buf, sem_a)
            cb = pltpu.async_copy(b_hbm.at[didx], b_buf, sem_b)
            ca.wait()
            cb.wait()

            def _row(r, _):
                for c in range(vpr):
                    s = pl.ds(c * LANES, LANES)
                    a_buf[r, s] = jnp.maximum(a_buf[r, s] + b_buf[r, s], 0.0)
                return 0

            lax.fori_loop(0, CH, _row, 0)
            pltpu.sync_copy(a_buf, acc.at[didx], add=True)
            return 0

        lax.fori_loop(0, nchunks, _chunk, 0)
        plsc.subcore_barrier()

        r0 = sid * rows_per_tile
        pltpu.sync_copy(acc.at[pl.ds(r0, rows_per_tile)],
                        out_hbm.at[cid, pl.ds(r0, rows_per_tile)])

    return sc_edges


def _merge_body(p_ref, o_ref):
    o_ref[...] = p_ref[0] + p_ref[1]


@functools.partial(jax.jit, static_argnames=("n", "blk"))
def _merge(partials, n, blk=1000):
    npad, hid = partials.shape[1], partials.shape[2]
    return pl.pallas_call(
        _merge_body,
        grid=(n // blk,),
        in_specs=[pl.BlockSpec((2, blk, hid), lambda i: (0, i, 0))],
        out_specs=pl.BlockSpec((blk, hid), lambda i: (i, 0)),
        out_shape=jax.ShapeDtypeStruct((n, hid), jnp.float32),
    )(partials)


def kernel(x, edge_index, W_in, b_in, W_msg, b_msg):
    n, d = x.shape
    hid = W_in.shape[1]
    e = edge_index.shape[1]

    a, b = _proj(x, W_in, b_in, W_msg[:hid], W_msg[hid:], b_msg)

    epw = -(-e // (NW * CH)) * CH
    e_pad = NW * epw
    src = edge_index[0]
    dst = edge_index[1]
    if e_pad > e:
        src = jnp.concatenate([src, jnp.zeros((e_pad - e,), jnp.int32)])
        dst = jnp.concatenate([dst, jnp.full((e_pad - e,), n, jnp.int32)])
    b_pad = jnp.concatenate([b, jnp.zeros((1, hid), jnp.float32)], axis=0)

    npad = -(-(n + 1) // (NS * CH)) * (NS * CH)
    sc_edges = _make_sc(epw, epw // CH, npad, hid)
    partials = sc_edges(a, b_pad, src, dst)

    return _merge(partials, n)
